# baseline (speedup 1.0000x reference)
import jax
import jax.numpy as jnp
from jax.experimental import pallas as pl
from jax.experimental.pallas import tpu as pltpu

_PATCH = 4
_NUM_CLASSES = 16


def kernel(x, w_patch, b_patch, w_fc_t, b_fc, w_proj_t, b_proj):
    B, _, T, F = x.shape
    patch = _PATCH
    Hp, Wp = F // patch, T // patch
    K = patch * patch
    G = B * Wp
    NC = _NUM_CLASSES
    C2 = w_patch.shape[1]
    C = C2 // 2
    C2ND = (3 * C) // 4
    BLK = C // 4
    BT = B * T
    R = Hp * G
    NB1 = C // BLK
    NB2 = C2ND // BLK
    NSTEP = NB1 + NB2

    jj = jnp.array([0] * NB1 + [1] * NB2, jnp.int32)

    def _sed_kernel(jj_ref, x_ref, wp_ref, bp_ref, wfc_ref, bfc_ref, wpr_ref,
                    bpr_ref, o_ref, patches_s, xs_s, z_s):
        i = pl.program_id(0)
        j = jj_ref[i]
        first0, last0 = i == 0, i == NB1 - 1
        first1, last1 = i == NB1, i == NSTEP - 1

        @pl.when(first0)
        def _build_patches():
            X = x_ref[...].reshape(BT, F)
            r_i = jax.lax.broadcasted_iota(jnp.int32, (R, BT), 0)
            c_i = jax.lax.broadcasted_iota(jnp.int32, (R, BT), 1)
            rf_i = jax.lax.broadcasted_iota(jnp.int32, (R, F), 0)
            cf_i = jax.lax.broadcasted_iota(jnp.int32, (R, F), 1)
            msk = (cf_i // patch) == (rf_i // G)
            rk = jax.lax.broadcasted_iota(jnp.int32, (F, K), 0)
            kk = jax.lax.broadcasted_iota(jnp.int32, (F, K), 1)
            acc = jnp.zeros((R, K), jnp.float32)
            for pt in range(patch):
                sel = (c_i == (r_i % G) * patch + pt).astype(jnp.float32)
                a = jnp.dot(sel, X, preferred_element_type=jnp.float32)
                a = jnp.where(msk, a, 0.0)
                cc = (kk == (rk % patch) * patch + pt).astype(jnp.float32)
                acc = acc + jnp.dot(a, cc, preferred_element_type=jnp.float32)
            patches_s[...] = acc

        @pl.when(jnp.logical_or(first0, first1))
        def _front_end():
            emb = jnp.maximum(
                jnp.dot(patches_s[...], wp_ref[...],
                        preferred_element_type=jnp.float32)
                + bp_ref[...], 0.0)
            xacc = emb[0:G, :]
            for h in range(1, Hp):
                xacc = xacc + emb[h * G:(h + 1) * G, :]
            xt = xacc * (1.0 / Hp)

            zrow = jnp.zeros((1, C), jnp.float32)
            x_prev = jnp.concatenate([zrow, xt[:-1, :]], axis=0)
            x_next = jnp.concatenate([xt[1:, :], zrow], axis=0)
            t_idx = jax.lax.broadcasted_iota(jnp.int32, (G, C), 0) % Wp
            first = t_idx == 0
            last = t_idx == Wp - 1
            x1 = jnp.maximum(xt,
                             jnp.maximum(jnp.where(first, -jnp.inf, x_prev),
                                         jnp.where(last, -jnp.inf, x_next)))
            x2 = (xt + jnp.where(first, 0.0, x_prev)
                  + jnp.where(last, 0.0, x_next)) * (1.0 / 3.0)
            xs_s[...] = x1 + x2
            z_s[...] = jnp.zeros((G, 4 * NC), jnp.float32)

        y = jnp.maximum(
            jnp.dot(xs_s[...], wfc_ref[...], preferred_element_type=jnp.float32)
            + bfc_ref[...], 0.0)
        z_s[...] = z_s[...] + jnp.dot(y, wpr_ref[...],
                                      preferred_element_type=jnp.float32)

        @pl.when(jnp.logical_or(last0, last1))
        def _tail():
            z = z_s[...] + bpr_ref[...]
            att_all = jnp.tanh(z[:, :2 * NC])
            cla_all = jax.nn.sigmoid(z[:, 2 * NC:])
            is0 = j == 0
            att = jnp.where(is0, att_all[:, :NC], att_all[:, NC:2 * NC])
            cla = jnp.where(is0, cla_all[:, :NC], cla_all[:, NC:2 * NC])

            preds = []
            for b in range(B):
                a_b = att[b * Wp:(b + 1) * Wp, :]
                c_b = cla[b * Wp:(b + 1) * Wp, :]
                m = jnp.max(a_b, axis=0, keepdims=True)
                e = jnp.exp(a_b - m)
                norm_att = e * pl.reciprocal(
                    jnp.sum(e, axis=0, keepdims=True), approx=True)
                clip = jnp.sum(norm_att * c_b, axis=0, keepdims=True)
                maxframe = jnp.max(c_b, axis=0, keepdims=True)
                preds.append(0.5 * (clip + maxframe))
            pred = jnp.concatenate(preds, axis=0)

            o_ref[...] = jnp.where(is0, 0.3 * pred, o_ref[...] + 0.7 * pred)

    pred = pl.pallas_call(
        _sed_kernel,
        out_shape=jax.ShapeDtypeStruct((B, NC), jnp.float32),
        grid_spec=pltpu.PrefetchScalarGridSpec(
            num_scalar_prefetch=1,
            grid=(NSTEP,),
            in_specs=[
                pl.BlockSpec((B, 1, T, F), lambda i, jj: (0, 0, 0, 0)),
                pl.BlockSpec((K, C), lambda i, jj: (0, jj[i])),
                pl.BlockSpec((1, C), lambda i, jj: (0, jj[i])),
                pl.BlockSpec((C, BLK), lambda i, jj: (jj[i], i)),
                pl.BlockSpec((1, BLK), lambda i, jj: (0, i)),
                pl.BlockSpec((BLK, 4 * NC), lambda i, jj: (i, 0)),
                pl.BlockSpec((1, 4 * NC), lambda i, jj: (0, 0)),
            ],
            out_specs=pl.BlockSpec((B, NC), lambda i, jj: (0, 0)),
            scratch_shapes=[
                pltpu.VMEM((R, K), jnp.float32),
                pltpu.VMEM((G, C), jnp.float32),
                pltpu.VMEM((G, 4 * NC), jnp.float32),
            ],
        ),
        compiler_params=pltpu.CompilerParams(
            dimension_semantics=("arbitrary",)),
    )(jj, x, w_patch, b_patch, w_fc_t, b_fc, w_proj_t, b_proj)

    return pred, pred


# speedup vs baseline: 1.0074x; 1.0074x over previous
import jax
import jax.numpy as jnp
from jax.experimental import pallas as pl
from jax.experimental.pallas import tpu as pltpu

_PATCH = 4
_NUM_CLASSES = 16


def kernel(x, w_patch, b_patch, w_fc_t, b_fc, w_proj_t, b_proj):
    B, _, T, F = x.shape
    patch = _PATCH
    Hp, Wp = F // patch, T // patch
    K = patch * patch
    G = B * Wp
    NC = _NUM_CLASSES
    C2 = w_patch.shape[1]
    C = C2 // 2
    C2ND = (3 * C) // 4
    BLK = C // 4
    BT = B * T
    R = Hp * G
    NB1 = C // BLK
    NB2 = C2ND // BLK
    NSTEP = NB1 + NB2

    jj = jnp.array([0] * NB1 + [1] * NB2, jnp.int32)

    def _sed_kernel(jj_ref, x_ref, wp_ref, bp_ref, wfc_ref, bfc_ref, wpr_ref,
                    bpr_ref, o_ref, patches_s, xs_s, yacc_s):
        i = pl.program_id(0)
        j = jj_ref[i]
        first0, last0 = i == 0, i == NB1 - 1
        first1, last1 = i == NB1, i == NSTEP - 1

        @pl.when(first0)
        def _build_patches():
            X = x_ref[...].reshape(BT, F)
            r_i = jax.lax.broadcasted_iota(jnp.int32, (R, BT), 0)
            c_i = jax.lax.broadcasted_iota(jnp.int32, (R, BT), 1)
            rf_i = jax.lax.broadcasted_iota(jnp.int32, (R, F), 0)
            cf_i = jax.lax.broadcasted_iota(jnp.int32, (R, F), 1)
            msk = (cf_i // patch) == (rf_i // G)
            rk = jax.lax.broadcasted_iota(jnp.int32, (F, K), 0)
            kk = jax.lax.broadcasted_iota(jnp.int32, (F, K), 1)
            acc = jnp.zeros((R, K), jnp.float32)
            for pt in range(patch):
                sel = (c_i == (r_i % G) * patch + pt).astype(jnp.float32)
                a = jnp.dot(sel, X, preferred_element_type=jnp.float32)
                a = jnp.where(msk, a, 0.0)
                cc = (kk == (rk % patch) * patch + pt).astype(jnp.float32)
                acc = acc + jnp.dot(a, cc, preferred_element_type=jnp.float32)
            patches_s[...] = acc

        @pl.when(jnp.logical_or(first0, first1))
        def _front_end():
            emb = jnp.maximum(
                jnp.dot(patches_s[...], wp_ref[...],
                        preferred_element_type=jnp.float32)
                + bp_ref[...], 0.0)
            xacc = emb[0:G, :]
            for h in range(1, Hp):
                xacc = xacc + emb[h * G:(h + 1) * G, :]
            xt = xacc * (1.0 / Hp)

            zrow = jnp.zeros((1, C), jnp.float32)
            x_prev = jnp.concatenate([zrow, xt[:-1, :]], axis=0)
            x_next = jnp.concatenate([xt[1:, :], zrow], axis=0)
            t_idx = jax.lax.broadcasted_iota(jnp.int32, (G, C), 0) % Wp
            first = t_idx == 0
            last = t_idx == Wp - 1
            x1 = jnp.maximum(xt,
                             jnp.maximum(jnp.where(first, -jnp.inf, x_prev),
                                         jnp.where(last, -jnp.inf, x_next)))
            x2 = (xt + jnp.where(first, 0.0, x_prev)
                  + jnp.where(last, 0.0, x_next)) * (1.0 / 3.0)
            xs = x1 + x2
            for a in range(NB1):
                xs_s[a] = xs[:, a * BLK:(a + 1) * BLK]
            yacc_s[...] = jnp.zeros((G, C), jnp.float32)

        a_loc = i - j * NB1
        yacc_s[...] = yacc_s[...] + jnp.dot(
            xs_s[a_loc], wfc_ref[...], preferred_element_type=jnp.float32)

        @pl.when(jnp.logical_or(last0, last1))
        def _tail():
            y = jnp.maximum(yacc_s[...] + bfc_ref[...], 0.0)
            z = (jnp.dot(y, wpr_ref[...], preferred_element_type=jnp.float32)
                 + bpr_ref[...])
            att_all = jnp.tanh(z[:, :2 * NC])
            cla_all = jax.nn.sigmoid(z[:, 2 * NC:])
            is0 = j == 0
            att = jnp.where(is0, att_all[:, :NC], att_all[:, NC:2 * NC])
            cla = jnp.where(is0, cla_all[:, :NC], cla_all[:, NC:2 * NC])

            preds = []
            for b in range(B):
                a_b = att[b * Wp:(b + 1) * Wp, :]
                c_b = cla[b * Wp:(b + 1) * Wp, :]
                m = jnp.max(a_b, axis=0, keepdims=True)
                e = jnp.exp(a_b - m)
                norm_att = e * pl.reciprocal(
                    jnp.sum(e, axis=0, keepdims=True), approx=True)
                clip = jnp.sum(norm_att * c_b, axis=0, keepdims=True)
                maxframe = jnp.max(c_b, axis=0, keepdims=True)
                preds.append(0.5 * (clip + maxframe))
            pred = jnp.concatenate(preds, axis=0)

            o_ref[...] = jnp.where(is0, 0.3 * pred, o_ref[...] + 0.7 * pred)

    pred = pl.pallas_call(
        _sed_kernel,
        out_shape=jax.ShapeDtypeStruct((B, NC), jnp.float32),
        grid_spec=pltpu.PrefetchScalarGridSpec(
            num_scalar_prefetch=1,
            grid=(NSTEP,),
            in_specs=[
                pl.BlockSpec((B, 1, T, F), lambda i, jj: (0, 0, 0, 0)),
                pl.BlockSpec((K, C), lambda i, jj: (0, jj[i])),
                pl.BlockSpec((1, C), lambda i, jj: (0, jj[i])),
                pl.BlockSpec((BLK, C), lambda i, jj: (i, jj[i])),
                pl.BlockSpec((1, C), lambda i, jj: (0, jj[i])),
                pl.BlockSpec((C, 4 * NC), lambda i, jj: (jj[i], 0)),
                pl.BlockSpec((1, 4 * NC), lambda i, jj: (0, 0)),
            ],
            out_specs=pl.BlockSpec((B, NC), lambda i, jj: (0, 0)),
            scratch_shapes=[
                pltpu.VMEM((R, K), jnp.float32),
                pltpu.VMEM((NB1, G, BLK), jnp.float32),
                pltpu.VMEM((G, C), jnp.float32),
            ],
        ),
        compiler_params=pltpu.CompilerParams(
            dimension_semantics=("arbitrary",)),
    )(jj, x, w_patch, b_patch, w_fc_t, b_fc, w_proj_t, b_proj)

    return pred, pred


# speedup vs baseline: 1.5121x; 1.5010x over previous
import jax
import jax.numpy as jnp
from jax.experimental import pallas as pl
from jax.experimental.pallas import tpu as pltpu

_PATCH = 4
_NUM_CLASSES = 16


def kernel(x, w_patch, b_patch, w_fc_t, b_fc, w_proj_t, b_proj):
    B, _, T, F = x.shape
    patch = _PATCH
    Hp, Wp = F // patch, T // patch
    K = patch * patch
    G = B * Wp
    NC = _NUM_CLASSES
    C2 = w_patch.shape[1]
    C = C2 // 2
    C2ND = (3 * C) // 4
    BT = B * T
    R = Hp * G

    def _sed_kernel(x_ref, wp_ref, bp_ref, wfc_hbm, bfc_ref, wpr_ref,
                    bpr_ref, o_ref, wfc1_s, wfc2_s, sem1, sem2):
        cp1 = pltpu.make_async_copy(
            wfc_hbm.at[pl.ds(0, C), pl.ds(0, C)], wfc1_s, sem1)
        cp1.start()
        cp2 = pltpu.make_async_copy(
            wfc_hbm.at[pl.ds(C, C2ND), pl.ds(C, C2ND)], wfc2_s, sem2)
        cp2.start()

        X = x_ref[...].reshape(BT, F)
        r_i = jax.lax.broadcasted_iota(jnp.int32, (R, BT), 0)
        c_i = jax.lax.broadcasted_iota(jnp.int32, (R, BT), 1)
        rf_i = jax.lax.broadcasted_iota(jnp.int32, (R, F), 0)
        cf_i = jax.lax.broadcasted_iota(jnp.int32, (R, F), 1)
        msk = (cf_i // patch) == (rf_i // G)
        rk = jax.lax.broadcasted_iota(jnp.int32, (F, K), 0)
        kk = jax.lax.broadcasted_iota(jnp.int32, (F, K), 1)
        patches = jnp.zeros((R, K), jnp.float32)
        for pt in range(patch):
            sel = (c_i == (r_i % G) * patch + pt).astype(jnp.float32)
            a = jnp.dot(sel, X, preferred_element_type=jnp.float32)
            a = jnp.where(msk, a, 0.0)
            cc = (kk == (rk % patch) * patch + pt).astype(jnp.float32)
            patches = patches + jnp.dot(a, cc,
                                        preferred_element_type=jnp.float32)

        emb = jnp.maximum(
            jnp.dot(patches, wp_ref[...], preferred_element_type=jnp.float32)
            + bp_ref[...], 0.0)

        xacc = emb[0:G, :]
        for h in range(1, Hp):
            xacc = xacc + emb[h * G:(h + 1) * G, :]
        xt = xacc * (1.0 / Hp)

        zrow = jnp.zeros((1, C2), jnp.float32)
        x_prev = jnp.concatenate([zrow, xt[:-1, :]], axis=0)
        x_next = jnp.concatenate([xt[1:, :], zrow], axis=0)
        t_idx = jax.lax.broadcasted_iota(jnp.int32, (G, C2), 0) % Wp
        first = t_idx == 0
        last = t_idx == Wp - 1
        x1 = jnp.maximum(xt, jnp.maximum(jnp.where(first, -jnp.inf, x_prev),
                                         jnp.where(last, -jnp.inf, x_next)))
        x2 = (xt + jnp.where(first, 0.0, x_prev)
              + jnp.where(last, 0.0, x_next)) * (1.0 / 3.0)
        xs = x1 + x2

        wpr = wpr_ref[...]
        cp1.wait()
        y1 = jnp.maximum(
            jnp.dot(xs[:, :C], wfc1_s[...], preferred_element_type=jnp.float32)
            + bfc_ref[:, :C], 0.0)
        z1 = jnp.dot(y1, wpr[:C, :], preferred_element_type=jnp.float32)
        cp2.wait()
        y2 = jnp.maximum(
            jnp.dot(xs[:, C:C + C2ND], wfc2_s[...],
                    preferred_element_type=jnp.float32)
            + bfc_ref[:, C:C + C2ND], 0.0)
        z2 = jnp.dot(y2, wpr[C:C + C2ND, :], preferred_element_type=jnp.float32)
        z = z1 + z2 + bpr_ref[...]

        att = jnp.tanh(z[:, :2 * NC])
        cla = jax.nn.sigmoid(z[:, 2 * NC:])

        preds = []
        for b in range(B):
            a_b = att[b * Wp:(b + 1) * Wp, :]
            c_b = cla[b * Wp:(b + 1) * Wp, :]
            m = jnp.max(a_b, axis=0, keepdims=True)
            e = jnp.exp(a_b - m)
            norm_att = e * pl.reciprocal(jnp.sum(e, axis=0, keepdims=True),
                                         approx=True)
            clip = jnp.sum(norm_att * c_b, axis=0, keepdims=True)
            maxframe = jnp.max(c_b, axis=0, keepdims=True)
            pred_b = 0.5 * (clip + maxframe)
            preds.append(0.3 * pred_b[:, :NC] + 0.7 * pred_b[:, NC:])
        o_ref[...] = jnp.concatenate(preds, axis=0)

    pred = pl.pallas_call(
        _sed_kernel,
        out_shape=jax.ShapeDtypeStruct((B, NC), jnp.float32),
        grid=(1,),
        in_specs=[
            pl.BlockSpec((B, 1, T, F), lambda i: (0, 0, 0, 0)),
            pl.BlockSpec((K, C2), lambda i: (0, 0)),
            pl.BlockSpec((1, C2), lambda i: (0, 0)),
            pl.BlockSpec(memory_space=pl.ANY),
            pl.BlockSpec((1, C2), lambda i: (0, 0)),
            pl.BlockSpec((C2, 4 * NC), lambda i: (0, 0)),
            pl.BlockSpec((1, 4 * NC), lambda i: (0, 0)),
        ],
        out_specs=pl.BlockSpec((B, NC), lambda i: (0, 0)),
        scratch_shapes=[
            pltpu.VMEM((C, C), jnp.float32),
            pltpu.VMEM((C2ND, C2ND), jnp.float32),
            pltpu.SemaphoreType.DMA,
            pltpu.SemaphoreType.DMA,
        ],
        compiler_params=pltpu.CompilerParams(
            dimension_semantics=("arbitrary",)),
    )(x, w_patch, b_patch, w_fc_t, b_fc, w_proj_t, b_proj)

    return pred, pred


# speedup vs baseline: 2.5099x; 1.6599x over previous
import jax
import jax.numpy as jnp
from jax.experimental import pallas as pl
from jax.experimental.pallas import tpu as pltpu

_PATCH = 4
_NUM_CLASSES = 16


def kernel(x, w_patch, b_patch, w_fc_t, b_fc, w_proj_t, b_proj):
    B, _, T, F = x.shape
    patch = _PATCH
    Hp, Wp = F // patch, T // patch
    K = patch * patch
    G = B * Wp
    NC = _NUM_CLASSES
    C2 = w_patch.shape[1]
    C = C2 // 2
    C2ND = (3 * C) // 4
    BT = B * T
    R = Hp * G

    def _sed_kernel(x_ref, wp_ref, bp_ref, wfc_hbm, bfc_ref, wprt_ref,
                    bpr_ref, o_ref, o2_ref, wfc1_s, wfc2_s, sem1, sem2):
        cp1 = pltpu.make_async_copy(
            wfc_hbm.at[pl.ds(0, C), pl.ds(0, C)], wfc1_s, sem1)
        cp1.start()
        cp2 = pltpu.make_async_copy(
            wfc_hbm.at[pl.ds(C, C2ND), pl.ds(C, C2ND)], wfc2_s, sem2)
        cp2.start()

        X = x_ref[...].reshape(BT, F)
        r_i = jax.lax.broadcasted_iota(jnp.int32, (R, BT), 0)
        c_i = jax.lax.broadcasted_iota(jnp.int32, (R, BT), 1)
        rf_i = jax.lax.broadcasted_iota(jnp.int32, (R, F), 0)
        cf_i = jax.lax.broadcasted_iota(jnp.int32, (R, F), 1)
        msk = (cf_i // patch) == (rf_i // G)
        rk = jax.lax.broadcasted_iota(jnp.int32, (F, K), 0)
        kk = jax.lax.broadcasted_iota(jnp.int32, (F, K), 1)
        patches = jnp.zeros((R, K), jnp.float32)
        for pt in range(patch):
            sel = (c_i == (r_i % G) * patch + pt).astype(jnp.float32)
            a = jnp.dot(sel, X, preferred_element_type=jnp.float32)
            a = jnp.where(msk, a, 0.0)
            cc = (kk == (rk % patch) * patch + pt).astype(jnp.float32)
            patches = patches + jnp.dot(a, cc,
                                        preferred_element_type=jnp.float32)

        emb = jnp.maximum(
            jnp.dot(patches, wp_ref[...], preferred_element_type=jnp.float32)
            + bp_ref[...], 0.0)

        xacc = emb[0:G, :]
        for h in range(1, Hp):
            xacc = xacc + emb[h * G:(h + 1) * G, :]
        xt = xacc * (1.0 / Hp)

        zrow = jnp.zeros((1, C2), jnp.float32)
        x_prev = jnp.concatenate([zrow, xt[:-1, :]], axis=0)
        x_next = jnp.concatenate([xt[1:, :], zrow], axis=0)
        t_idx = jax.lax.broadcasted_iota(jnp.int32, (G, C2), 0) % Wp
        first = t_idx == 0
        last = t_idx == Wp - 1
        x1 = jnp.maximum(xt, jnp.maximum(jnp.where(first, -jnp.inf, x_prev),
                                         jnp.where(last, -jnp.inf, x_next)))
        x2 = (xt + jnp.where(first, 0.0, x_prev)
              + jnp.where(last, 0.0, x_next)) * (1.0 / 3.0)
        xs = x1 + x2

        wprt = wprt_ref[...]
        dn = (((1,), (1,)), ((), ()))
        cp1.wait()
        y1 = jnp.maximum(
            jnp.dot(xs[:, :C], wfc1_s[...], preferred_element_type=jnp.float32)
            + bfc_ref[:, :C], 0.0)
        z1 = jax.lax.dot_general(y1, wprt[:, :C], dn,
                                 preferred_element_type=jnp.float32)
        cp2.wait()
        y2 = jnp.maximum(
            jnp.dot(xs[:, C:C + C2ND], wfc2_s[...],
                    preferred_element_type=jnp.float32)
            + bfc_ref[:, C:C + C2ND], 0.0)
        z2 = jax.lax.dot_general(y2, wprt[:, C:C + C2ND], dn,
                                 preferred_element_type=jnp.float32)
        z = z1 + z2 + bpr_ref[...]

        att = jnp.tanh(z[:, :2 * NC])
        cla = jax.nn.sigmoid(z[:, 2 * NC:])

        preds = []
        for b in range(B):
            a_b = att[b * Wp:(b + 1) * Wp, :]
            c_b = cla[b * Wp:(b + 1) * Wp, :]
            m = jnp.max(a_b, axis=0, keepdims=True)
            e = jnp.exp(a_b - m)
            norm_att = e * pl.reciprocal(jnp.sum(e, axis=0, keepdims=True),
                                         approx=True)
            clip = jnp.sum(norm_att * c_b, axis=0, keepdims=True)
            maxframe = jnp.max(c_b, axis=0, keepdims=True)
            pred_b = 0.5 * (clip + maxframe)
            preds.append(0.3 * pred_b[:, :NC] + 0.7 * pred_b[:, NC:])
        pred = jnp.concatenate(preds, axis=0)
        o_ref[...] = pred
        o2_ref[...] = pred

    pred, pred2 = pl.pallas_call(
        _sed_kernel,
        out_shape=(jax.ShapeDtypeStruct((B, NC), jnp.float32),
                   jax.ShapeDtypeStruct((B, NC), jnp.float32)),
        grid=(1,),
        in_specs=[
            pl.BlockSpec((B, 1, T, F), lambda i: (0, 0, 0, 0)),
            pl.BlockSpec((K, C2), lambda i: (0, 0)),
            pl.BlockSpec((1, C2), lambda i: (0, 0)),
            pl.BlockSpec(memory_space=pl.ANY),
            pl.BlockSpec((1, C2), lambda i: (0, 0)),
            pl.BlockSpec((4 * NC, C2), lambda i: (0, 0)),
            pl.BlockSpec((1, 4 * NC), lambda i: (0, 0)),
        ],
        out_specs=(pl.BlockSpec((B, NC), lambda i: (0, 0)),
                   pl.BlockSpec((B, NC), lambda i: (0, 0))),
        scratch_shapes=[
            pltpu.VMEM((C, C), jnp.float32),
            pltpu.VMEM((C2ND, C2ND), jnp.float32),
            pltpu.SemaphoreType.DMA,
            pltpu.SemaphoreType.DMA,
        ],
        compiler_params=pltpu.CompilerParams(
            dimension_semantics=("arbitrary",)),
    )(x, w_patch, b_patch, w_fc_t, b_fc, w_proj_t.T, b_proj)

    return pred, pred2
